# baseline (speedup 1.0000x reference)
import functools

import jax
import jax.numpy as jnp
from jax import lax
from jax.experimental import pallas as pl
from jax.experimental.pallas import tpu as pltpu
from jax.experimental.pallas import tpu_sc as plsc

NUM_CODES = 8192
DIM = 32
TILE_M = 4096

_NC = 2
_NS = 16
_NW = _NC * _NS
_CHUNK = 128


_SUB = 512
_N_SUB = TILE_M // _SUB


def _vq_kernel(z_ref, emb_ref, idx_ref, en_ref):
    @pl.when(pl.program_id(0) == 0)
    def _():
        e = emb_ref[...]
        nn = jnp.sqrt(jnp.sum(e * e, axis=1, keepdims=True))
        en_ref[...] = e / jnp.maximum(nn, 1e-12)

    xt = z_ref[0]
    en = en_ref[...]
    parts = []
    for s in range(_N_SUB):
        x = xt[:, s * _SUB:(s + 1) * _SUB].T
        n = jnp.sqrt(jnp.sum(x * x, axis=1, keepdims=True))
        ez = x / jnp.maximum(n, 1e-12)
        sim = jax.lax.dot_general(ez, en, (((1,), (1,)), ((), ())))
        parts.append(jnp.argmax(sim, axis=1).astype(jnp.int32))
    idx_ref[0, 0, :] = jnp.concatenate(parts)


def _make_sc_gather(b_total):
    b_per_w = b_total // _NW
    n_chunks = b_per_w // _CHUNK
    mesh = plsc.VectorSubcoreMesh(core_axis_name="c", subcore_axis_name="s")

    @functools.partial(
        pl.kernel,
        mesh=mesh,
        compiler_params=pltpu.CompilerParams(use_tc_tiling_on_sc=False),
        out_type=jax.ShapeDtypeStruct((b_total, DIM), jnp.float32),
        scratch_types=[
            pltpu.VMEM((n_chunks, _CHUNK), jnp.int32),
            pltpu.VMEM((b_per_w, DIM), jnp.float32),
            pltpu.SemaphoreType.DMA,
        ],
    )
    def sc_gather(table_hbm, idx_hbm, out_hbm, idx_v, rows_v, sem):
        wid = lax.axis_index("s") * _NC + lax.axis_index("c")
        base = wid * b_per_w
        pltpu.sync_copy(idx_hbm.at[wid], idx_v)
        for j in range(n_chunks):
            pltpu.async_copy(
                table_hbm.at[idx_v.at[j]],
                rows_v.at[pl.ds(j * _CHUNK, _CHUNK)],
                sem,
            )
        for j in range(n_chunks):
            pltpu.make_async_copy(
                table_hbm.at[idx_v.at[j]],
                rows_v.at[pl.ds(j * _CHUNK, _CHUNK)],
                sem,
            ).wait()
        pltpu.sync_copy(rows_v, out_hbm.at[pl.ds(base, b_per_w)])

    return sc_gather


def _vq_idx(z_part, embedding):
    bp, d, tq = z_part.shape
    mp = bp * tq
    t_tiles = tq // TILE_M
    return pl.pallas_call(
        _vq_kernel,
        grid=(mp // TILE_M,),
        in_specs=[
            pl.BlockSpec((1, DIM, TILE_M),
                         lambda i: (i // t_tiles, 0, i % t_tiles)),
            pl.BlockSpec((NUM_CODES, DIM), lambda i: (0, 0)),
        ],
        out_specs=pl.BlockSpec((1, 1, TILE_M), lambda i: (i, 0, 0)),
        out_shape=jax.ShapeDtypeStruct((mp // TILE_M, 1, TILE_M), jnp.int32),
        scratch_shapes=[pltpu.VMEM((NUM_CODES, DIM), jnp.float32)],
    )(z_part, embedding)


def kernel(z, embedding):
    b, d, tq = z.shape
    m = b * tq

    idx_flat = _vq_idx(z, embedding).reshape(m)
    idx_w = idx_flat.reshape(_NW, m // (_NW * _CHUNK), _CHUNK)
    zqf = _make_sc_gather(m)(embedding, idx_w)

    idx = idx_flat.reshape(b, tq)
    z_q = jnp.transpose(zqf.reshape(b, tq, d), (0, 2, 1))
    z_q_st = z + jax.lax.stop_gradient(z_q - z)
    return (z_q_st, idx, z_q)


# speedup vs baseline: 1.0082x; 1.0082x over previous
import functools

import jax
import jax.numpy as jnp
from jax import lax
from jax.experimental import pallas as pl
from jax.experimental.pallas import tpu as pltpu
from jax.experimental.pallas import tpu_sc as plsc

NUM_CODES = 8192
DIM = 32
TILE_M = 4096

_NC = 2
_NS = 16
_NW = _NC * _NS
_CHUNK = 128


_SUB = 128
_N_SUB = TILE_M // _SUB


def _vq_kernel(z_ref, emb_ref, idx_ref, en_ref):
    @pl.when(pl.program_id(0) == 0)
    def _():
        e = emb_ref[...]
        nn = jnp.sqrt(jnp.sum(e * e, axis=1, keepdims=True))
        en_ref[...] = e / jnp.maximum(nn, 1e-12)

    xt = z_ref[0]
    en = en_ref[...]
    parts = []
    for s in range(_N_SUB):
        x = xt[:, s * _SUB:(s + 1) * _SUB].T
        n = jnp.sqrt(jnp.sum(x * x, axis=1, keepdims=True))
        ez = x / jnp.maximum(n, 1e-12)
        sim = jax.lax.dot_general(ez, en, (((1,), (1,)), ((), ())))
        parts.append(jnp.argmax(sim, axis=1).astype(jnp.int32))
    idx_ref[0, 0, :] = jnp.concatenate(parts)


def _make_sc_gather(b_total):
    b_per_w = b_total // _NW
    n_chunks = b_per_w // _CHUNK
    mesh = plsc.VectorSubcoreMesh(core_axis_name="c", subcore_axis_name="s")

    @functools.partial(
        pl.kernel,
        mesh=mesh,
        compiler_params=pltpu.CompilerParams(use_tc_tiling_on_sc=False),
        out_type=jax.ShapeDtypeStruct((b_total, DIM), jnp.float32),
        scratch_types=[
            pltpu.VMEM((n_chunks, _CHUNK), jnp.int32),
            pltpu.VMEM((b_per_w, DIM), jnp.float32),
            pltpu.SemaphoreType.DMA,
        ],
    )
    def sc_gather(table_hbm, idx_hbm, out_hbm, idx_v, rows_v, sem):
        wid = lax.axis_index("s") * _NC + lax.axis_index("c")
        base = wid * b_per_w
        pltpu.sync_copy(idx_hbm.at[wid], idx_v)
        for j in range(n_chunks):
            pltpu.async_copy(
                table_hbm.at[idx_v.at[j]],
                rows_v.at[pl.ds(j * _CHUNK, _CHUNK)],
                sem,
            )
        for j in range(n_chunks):
            pltpu.make_async_copy(
                table_hbm.at[idx_v.at[j]],
                rows_v.at[pl.ds(j * _CHUNK, _CHUNK)],
                sem,
            ).wait()
        pltpu.sync_copy(rows_v, out_hbm.at[pl.ds(base, b_per_w)])

    return sc_gather


def _vq_idx(z_part, embedding):
    bp, d, tq = z_part.shape
    mp = bp * tq
    t_tiles = tq // TILE_M
    return pl.pallas_call(
        _vq_kernel,
        grid=(mp // TILE_M,),
        in_specs=[
            pl.BlockSpec((1, DIM, TILE_M),
                         lambda i: (i // t_tiles, 0, i % t_tiles)),
            pl.BlockSpec((NUM_CODES, DIM), lambda i: (0, 0)),
        ],
        out_specs=pl.BlockSpec((1, 1, TILE_M), lambda i: (i, 0, 0)),
        out_shape=jax.ShapeDtypeStruct((mp // TILE_M, 1, TILE_M), jnp.int32),
        scratch_shapes=[pltpu.VMEM((NUM_CODES, DIM), jnp.float32)],
    )(z_part, embedding)


def kernel(z, embedding):
    b, d, tq = z.shape
    m = b * tq

    idx_flat = _vq_idx(z, embedding).reshape(m)
    idx_w = idx_flat.reshape(_NW, m // (_NW * _CHUNK), _CHUNK)
    zqf = _make_sc_gather(m)(embedding, idx_w)

    idx = idx_flat.reshape(b, tq)
    z_q = jnp.transpose(zqf.reshape(b, tq, d), (0, 2, 1))
    z_q_st = z + jax.lax.stop_gradient(z_q - z)
    return (z_q_st, idx, z_q)


# speedup vs baseline: 1.0111x; 1.0029x over previous
import functools

import jax
import jax.numpy as jnp
from jax import lax
from jax.experimental import pallas as pl
from jax.experimental.pallas import tpu as pltpu
from jax.experimental.pallas import tpu_sc as plsc

NUM_CODES = 8192
DIM = 32
TILE_M = 4096

_NC = 2
_NS = 16
_NW = _NC * _NS
_CHUNK = 128


_SUB = 256
_N_SUB = TILE_M // _SUB


def _vq_kernel(z_ref, emb_ref, idx_ref, en_ref):
    @pl.when(pl.program_id(0) == 0)
    def _():
        e = emb_ref[...]
        nn = jnp.sqrt(jnp.sum(e * e, axis=1, keepdims=True))
        en_ref[...] = e / jnp.maximum(nn, 1e-12)

    xt = z_ref[0]
    en = en_ref[...]
    parts = []
    for s in range(_N_SUB):
        x = xt[:, s * _SUB:(s + 1) * _SUB].T
        n = jnp.sqrt(jnp.sum(x * x, axis=1, keepdims=True))
        ez = x / jnp.maximum(n, 1e-12)
        sim = jax.lax.dot_general(ez, en, (((1,), (1,)), ((), ())))
        parts.append(jnp.argmax(sim, axis=1).astype(jnp.int32))
    idx_ref[0, 0, :] = jnp.concatenate(parts)


def _make_sc_gather(b_total):
    b_per_w = b_total // _NW
    n_chunks = b_per_w // _CHUNK
    mesh = plsc.VectorSubcoreMesh(core_axis_name="c", subcore_axis_name="s")

    @functools.partial(
        pl.kernel,
        mesh=mesh,
        compiler_params=pltpu.CompilerParams(use_tc_tiling_on_sc=False),
        out_type=jax.ShapeDtypeStruct((b_total, DIM), jnp.float32),
        scratch_types=[
            pltpu.VMEM((n_chunks, _CHUNK), jnp.int32),
            pltpu.VMEM((b_per_w, DIM), jnp.float32),
            pltpu.SemaphoreType.DMA,
        ],
    )
    def sc_gather(table_hbm, idx_hbm, out_hbm, idx_v, rows_v, sem):
        wid = lax.axis_index("s") * _NC + lax.axis_index("c")
        base = wid * b_per_w
        pltpu.sync_copy(idx_hbm.at[wid], idx_v)
        for j in range(n_chunks):
            pltpu.async_copy(
                table_hbm.at[idx_v.at[j]],
                rows_v.at[pl.ds(j * _CHUNK, _CHUNK)],
                sem,
            )
        for j in range(n_chunks):
            pltpu.make_async_copy(
                table_hbm.at[idx_v.at[j]],
                rows_v.at[pl.ds(j * _CHUNK, _CHUNK)],
                sem,
            ).wait()
        pltpu.sync_copy(rows_v, out_hbm.at[pl.ds(base, b_per_w)])

    return sc_gather


def _vq_idx(z_part, embedding):
    bp, d, tq = z_part.shape
    mp = bp * tq
    t_tiles = tq // TILE_M
    return pl.pallas_call(
        _vq_kernel,
        grid=(mp // TILE_M,),
        in_specs=[
            pl.BlockSpec((1, DIM, TILE_M),
                         lambda i: (i // t_tiles, 0, i % t_tiles)),
            pl.BlockSpec((NUM_CODES, DIM), lambda i: (0, 0)),
        ],
        out_specs=pl.BlockSpec((1, 1, TILE_M), lambda i: (i, 0, 0)),
        out_shape=jax.ShapeDtypeStruct((mp // TILE_M, 1, TILE_M), jnp.int32),
        scratch_shapes=[pltpu.VMEM((NUM_CODES, DIM), jnp.float32)],
    )(z_part, embedding)


def kernel(z, embedding):
    b, d, tq = z.shape
    m = b * tq

    idx_flat = _vq_idx(z, embedding).reshape(m)
    idx_w = idx_flat.reshape(_NW, m // (_NW * _CHUNK), _CHUNK)
    zqf = _make_sc_gather(m)(embedding, idx_w)

    idx = idx_flat.reshape(b, tq)
    z_q = jnp.transpose(zqf.reshape(b, tq, d), (0, 2, 1))
    z_q_st = z + jax.lax.stop_gradient(z_q - z)
    return (z_q_st, idx, z_q)
